# baseline (speedup 1.0000x reference)
import functools

import jax
import jax.numpy as jnp
from jax import lax
from jax.experimental import pallas as pl
from jax.experimental.pallas import tpu as pltpu
from jax.experimental.pallas import tpu_sc as plsc

_B = 256
_N = 128
_H = 8
_V = 129
_PLANE = _N * _N
_CHUNK = 1024
_VECS = _CHUNK // 16
_NCHUNK = _PLANE // _CHUNK
_NBUF = 6


@functools.cache
def _build_sc_kernel():
    info = plsc.get_sparse_core_info()
    nc, ns = info.num_cores, info.num_subcores
    nw = nc * ns
    bpw = _B // nw
    nst = bpw * _NCHUNK
    mesh = plsc.VectorSubcoreMesh(core_axis_name="c", subcore_axis_name="s")

    @functools.partial(
        pl.kernel,
        mesh=mesh,
        out_type=jax.ShapeDtypeStruct((_B * _H * _PLANE,), jnp.float32),
        compiler_params=pltpu.CompilerParams(needs_layout_passes=False),
        scratch_types=[
            pltpu.VMEM((_V, _H), jnp.float32),
            pltpu.VMEM((_H * _V,), jnp.float32),
        ]
        + [pltpu.VMEM((_CHUNK,), jnp.int32) for _ in range(_NBUF)]
        + [pltpu.VMEM((_H, _CHUNK), jnp.float32) for _ in range(_NBUF)]
        + [pltpu.SemaphoreType.DMA for _ in range(2 * _NBUF)],
    )
    def sc_kernel(dm_hbm, tab_hbm, out_hbm, tabsrc_v, tab_v, *bufs):
        idx_bufs = bufs[:_NBUF]
        out_bufs = bufs[_NBUF:2 * _NBUF]
        in_sems = bufs[2 * _NBUF:3 * _NBUF]
        out_sems = bufs[3 * _NBUF:4 * _NBUF]
        wid = lax.axis_index("s") * nc + lax.axis_index("c")
        pltpu.sync_copy(tab_hbm, tabsrc_v)
        lanes = lax.iota(jnp.int32, 16)
        for h in range(_H):
            hvec = jnp.full((16,), h, jnp.int32)
            for dv in range((_V + 15) // 16):
                dvec = lanes + dv * 16
                msk = dvec < _V
                dsafe = jnp.where(msk, dvec, 0)
                val = plsc.load_gather(tabsrc_v, [dsafe, hvec])
                plsc.store_scatter(tab_v, [dvec + h * _V], val, mask=msk)
        in_base = wid * (bpw * _PLANE)
        out_base = wid * (bpw * _H * _PLANE)

        def in_desc(t, buf):
            return pltpu.make_async_copy(
                dm_hbm.at[pl.ds(in_base + t * _CHUNK, _CHUNK)],
                idx_bufs[buf],
                in_sems[buf],
            )

        def out_descs(t, buf):
            off = out_base + (t // _NCHUNK) * (_H * _PLANE) + (t % _NCHUNK) * _CHUNK
            return [
                pltpu.make_async_copy(
                    out_bufs[buf].at[h],
                    out_hbm.at[pl.ds(off + h * _PLANE, _CHUNK)],
                    out_sems[buf],
                )
                for h in range(_H)
            ]

        def compute(buf):
            @plsc.parallel_loop(0, _VECS, unroll=2)
            def body(v):
                idx = idx_bufs[buf][pl.ds(v * 16, 16)]
                for h in range(_H):
                    val = plsc.load_gather(tab_v, [idx + h * _V])
                    out_bufs[buf][h, pl.ds(v * 16, 16)] = val

        for t in range(_NBUF):
            in_desc(t, t).start()
        for t in range(_NBUF):
            in_desc(t, t).wait()
            compute(t)
            for d in out_descs(t, t):
                d.start()
            in_desc(t + _NBUF, t).start()

        def group(k, carry):
            for buf in range(_NBUF):
                t = _NBUF * k + buf
                in_desc(t, buf).wait()
                for d in out_descs(t - _NBUF, buf):
                    d.wait()
                compute(buf)
                for d in out_descs(t, buf):
                    d.start()
                in_desc(t + _NBUF, buf).start()
            return carry

        ksteady = nst // _NBUF - 1
        lax.fori_loop(1, ksteady, group, 0)

        for t in range(_NBUF * ksteady, nst):
            buf = t % _NBUF
            in_desc(t, buf).wait()
            for d in out_descs(t - _NBUF, buf):
                d.wait()
            compute(buf)
            for d in out_descs(t, buf):
                d.start()
            if t + _NBUF < nst:
                in_desc(t + _NBUF, buf).start()
        for t in range(nst - _NBUF, nst):
            for d in out_descs(t, t % _NBUF):
                d.wait()

    return sc_kernel


def kernel(distance_matrix, distance_embedding):
    dm = distance_matrix
    if dm.dtype != jnp.int32:
        dm = dm.astype(jnp.int32)
    out = _build_sc_kernel()(dm.reshape(-1), distance_embedding)
    return out.reshape(_B, _H, _N, _N)


# speedup vs baseline: 1.1242x; 1.1242x over previous
import functools

import jax
import jax.numpy as jnp
from jax import lax
from jax.experimental import pallas as pl
from jax.experimental.pallas import tpu as pltpu
from jax.experimental.pallas import tpu_sc as plsc

_B = 256
_N = 128
_H = 8
_V = 129
_PLANE = _N * _N
_CHUNK = 1024
_VECS = _CHUNK // 16
_NCHUNK = _PLANE // _CHUNK
_NBUF = 4


@functools.cache
def _build_sc_kernel():
    info = plsc.get_sparse_core_info()
    nc, ns = info.num_cores, info.num_subcores
    nw = nc * ns
    bpw = _B // nw
    nst = bpw * _NCHUNK
    mesh = plsc.VectorSubcoreMesh(core_axis_name="c", subcore_axis_name="s")

    @functools.partial(
        pl.kernel,
        mesh=mesh,
        out_type=jax.ShapeDtypeStruct((_B * _H * _PLANE,), jnp.float32),
        compiler_params=pltpu.CompilerParams(needs_layout_passes=False),
        scratch_types=[
            pltpu.VMEM((_V, _H), jnp.float32),
            pltpu.VMEM((_H * _V,), jnp.float32),
        ]
        + [pltpu.VMEM((_CHUNK,), jnp.int32) for _ in range(_NBUF)]
        + [pltpu.VMEM((_H, _CHUNK), jnp.float32) for _ in range(_NBUF)]
        + [pltpu.SemaphoreType.DMA for _ in range(2 * _NBUF)],
    )
    def sc_kernel(dm_hbm, tab_hbm, out_hbm, tabsrc_v, tab_v, *bufs):
        idx_bufs = bufs[:_NBUF]
        out_bufs = bufs[_NBUF:2 * _NBUF]
        in_sems = bufs[2 * _NBUF:3 * _NBUF]
        out_sems = bufs[3 * _NBUF:4 * _NBUF]
        wid = lax.axis_index("s") * nc + lax.axis_index("c")
        pltpu.sync_copy(tab_hbm, tabsrc_v)
        lanes = lax.iota(jnp.int32, 16)
        for h in range(_H):
            hvec = jnp.full((16,), h, jnp.int32)
            for dv in range((_V + 15) // 16):
                dvec = lanes + dv * 16
                msk = dvec < _V
                dsafe = jnp.where(msk, dvec, 0)
                val = plsc.load_gather(tabsrc_v, [dsafe, hvec])
                plsc.store_scatter(tab_v, [dvec + h * _V], val, mask=msk)
        in_base = wid * (bpw * _PLANE)
        out_base = wid * (bpw * _H * _PLANE)

        def in_desc(t, buf):
            return pltpu.make_async_copy(
                dm_hbm.at[pl.ds(in_base + t * _CHUNK, _CHUNK)],
                idx_bufs[buf],
                in_sems[buf],
            )

        def out_descs(t, buf):
            off = out_base + (t // _NCHUNK) * (_H * _PLANE) + (t % _NCHUNK) * _CHUNK
            return [
                pltpu.make_async_copy(
                    out_bufs[buf].at[h],
                    out_hbm.at[pl.ds(off + h * _PLANE, _CHUNK)],
                    out_sems[buf],
                )
                for h in range(_H)
            ]

        def compute(buf):
            @plsc.parallel_loop(0, _VECS, unroll=2)
            def body(v):
                idx = idx_bufs[buf][pl.ds(v * 16, 16)]
                for h in range(_H):
                    val = plsc.load_gather(tab_v, [idx + h * _V])
                    out_bufs[buf][h, pl.ds(v * 16, 16)] = val

        for t in range(_NBUF):
            in_desc(t, t).start()
        for t in range(_NBUF):
            in_desc(t, t).wait()
            compute(t)
            for d in out_descs(t, t):
                d.start()
            in_desc(t + _NBUF, t).start()

        def group(k, carry):
            for buf in range(_NBUF):
                t = _NBUF * k + buf
                in_desc(t, buf).wait()
                for d in out_descs(t - _NBUF, buf):
                    d.wait()
                compute(buf)
                for d in out_descs(t, buf):
                    d.start()
                in_desc(t + _NBUF, buf).start()
            return carry

        ksteady = nst // _NBUF - 1
        lax.fori_loop(1, ksteady, group, 0)

        for t in range(_NBUF * ksteady, nst):
            buf = t % _NBUF
            in_desc(t, buf).wait()
            for d in out_descs(t - _NBUF, buf):
                d.wait()
            compute(buf)
            for d in out_descs(t, buf):
                d.start()
            if t + _NBUF < nst:
                in_desc(t + _NBUF, buf).start()
        for t in range(nst - _NBUF, nst):
            for d in out_descs(t, t % _NBUF):
                d.wait()

    return sc_kernel


def kernel(distance_matrix, distance_embedding):
    dm = distance_matrix
    if dm.dtype != jnp.int32:
        dm = dm.astype(jnp.int32)
    out = _build_sc_kernel()(dm.reshape(-1), distance_embedding)
    return out.reshape(_B, _H, _N, _N)
